# baseline (speedup 1.0000x reference)
import jax
import jax.numpy as jnp
from jax.experimental import pallas as pl
from jax.experimental.pallas import tpu as pltpu

_NUM_SLOTS = 8
_RANK = 64
_TOKENS = 4096
_D_IN = 2048
_D_OUT = 4096
_BT = 256
_W = 2 * _RANK
_LOG_BT = 8


def _lora_block_kernel(slot_smem, x_ref, a_ref, b_ref, o_ref, inter2):
    i = pl.program_id(0)
    a_bf = a_ref
    b_bf = b_ref
    base = i * _BT
    e_lo = slot_smem[base]
    e_hi = slot_smem[base + _BT - 1]
    x = x_ref[...].astype(jnp.bfloat16)
    rows = jax.lax.broadcasted_iota(jnp.int32, (_BT, 1), 0)

    def lower_bound(v):
        def step(_, lohi):
            lo, hi = lohi
            mid = (lo + hi) // 2
            below = slot_smem[base + mid] < v
            return (jnp.where(below, mid + 1, lo), jnp.where(below, hi, mid))

        lo, _ = jax.lax.fori_loop(0, _LOG_BT, step, (0, _BT))
        return lo

    @pl.when(e_hi - e_lo <= 1)
    def _pair_path():
        w = jnp.minimum(e_lo, _NUM_SLOTS - 2)
        t = lower_bound(w + 1)
        i0 = jnp.dot(x, a_bf[w], preferred_element_type=jnp.float32)
        m0 = (rows < t).astype(jnp.float32)
        inter2[:, 0:_RANK] = (i0 * m0).astype(jnp.bfloat16)
        i1 = jnp.dot(x, a_bf[w + 1], preferred_element_type=jnp.float32)
        m1 = (rows >= t).astype(jnp.float32)
        inter2[:, _RANK:_W] = (i1 * m1).astype(jnp.bfloat16)
        bwin = b_bf[pl.ds(w * _RANK, _W), :]
        o_ref[...] = jnp.dot(inter2[...], bwin,
                             preferred_element_type=jnp.float32)

    @pl.when(e_hi - e_lo > 1)
    def _multi_slot():
        def body(e, acc):
            inter = jnp.dot(x, a_bf[e], preferred_element_type=jnp.float32)
            lb = lower_bound(e)
            ub = lower_bound(e + 1)
            mask = ((rows >= lb) & (rows < ub)).astype(jnp.float32)
            inter = (inter * mask).astype(jnp.bfloat16)
            be = b_bf[pl.ds(e * _RANK, _RANK), :]
            return acc + jnp.dot(inter, be, preferred_element_type=jnp.float32)

        o_ref[...] = jax.lax.fori_loop(
            e_lo, e_hi + 1, body, jnp.zeros((_BT, _D_OUT), jnp.float32)
        )


def kernel(x, lora_a, lora_b, slot_ids):
    slot_ids = slot_ids.astype(jnp.int32)
    a_bf16 = lora_a.astype(jnp.bfloat16)
    b2d = lora_b.reshape(_NUM_SLOTS * _RANK, _D_OUT).astype(jnp.bfloat16)
    grid_spec = pltpu.PrefetchScalarGridSpec(
        num_scalar_prefetch=1,
        grid=(_TOKENS // _BT,),
        in_specs=[
            pl.BlockSpec((_BT, _D_IN), lambda i, s: (i, 0)),
            pl.BlockSpec((_NUM_SLOTS, _D_IN, _RANK), lambda i, s: (0, 0, 0)),
            pl.BlockSpec((_NUM_SLOTS * _RANK, _D_OUT), lambda i, s: (0, 0)),
        ],
        out_specs=pl.BlockSpec((_BT, _D_OUT), lambda i, s: (i, 0)),
        scratch_shapes=[
            pltpu.VMEM((_BT, _W), jnp.bfloat16),
        ],
    )
    return pl.pallas_call(
        _lora_block_kernel,
        grid_spec=grid_spec,
        out_shape=jax.ShapeDtypeStruct((_TOKENS, _D_OUT), jnp.float32),
    )(slot_ids, x, a_bf16, b2d)


# speedup vs baseline: 1.1343x; 1.1343x over previous
import jax
import jax.numpy as jnp
from jax.experimental import pallas as pl
from jax.experimental.pallas import tpu as pltpu

_NUM_SLOTS = 8
_RANK = 64
_TOKENS = 4096
_D_IN = 2048
_D_OUT = 4096
_BT = 256
_W = 2 * _RANK
_LOG_BT = 8


_DN_T = (((1,), (1,)), ((), ()))


def _lora_block_kernel(slot_smem, x_ref, at_ref, b_ref, o_ref, b_bf, inter2):
    i = pl.program_id(0)

    @pl.when(i == 0)
    def _cast_b():
        b_bf[...] = b_ref[...].astype(jnp.bfloat16)

    base = i * _BT
    e_lo = slot_smem[base]
    e_hi = slot_smem[base + _BT - 1]
    x = x_ref[...].astype(jnp.bfloat16)
    rows = jax.lax.broadcasted_iota(jnp.int32, (_BT, 1), 0)

    def lower_bound(v):
        def step(_, lohi):
            lo, hi = lohi
            mid = (lo + hi) // 2
            below = slot_smem[base + mid] < v
            return (jnp.where(below, mid + 1, lo), jnp.where(below, hi, mid))

        lo, _ = jax.lax.fori_loop(0, _LOG_BT, step, (0, _BT))
        return lo

    @pl.when(e_hi - e_lo <= 1)
    def _pair_path():
        w = jnp.minimum(e_lo, _NUM_SLOTS - 2)
        t = lower_bound(w + 1)
        i0 = jax.lax.dot_general(x, at_ref[w], _DN_T,
                                 preferred_element_type=jnp.float32)
        m0 = (rows < t).astype(jnp.float32)
        inter2[:, 0:_RANK] = (i0 * m0).astype(jnp.bfloat16)
        i1 = jax.lax.dot_general(x, at_ref[w + 1], _DN_T,
                                 preferred_element_type=jnp.float32)
        m1 = (rows >= t).astype(jnp.float32)
        inter2[:, _RANK:_W] = (i1 * m1).astype(jnp.bfloat16)
        bwin = b_bf[pl.ds(w * _RANK, _W), :]
        o_ref[...] = jnp.dot(inter2[...], bwin,
                             preferred_element_type=jnp.float32)

    @pl.when(e_hi - e_lo > 1)
    def _multi_slot():
        def body(e, acc):
            inter = jax.lax.dot_general(x, at_ref[e], _DN_T,
                                        preferred_element_type=jnp.float32)
            lb = lower_bound(e)
            ub = lower_bound(e + 1)
            mask = ((rows >= lb) & (rows < ub)).astype(jnp.float32)
            inter = (inter * mask).astype(jnp.bfloat16)
            be = b_bf[pl.ds(e * _RANK, _RANK), :]
            return acc + jnp.dot(inter, be, preferred_element_type=jnp.float32)

        o_ref[...] = jax.lax.fori_loop(
            e_lo, e_hi + 1, body, jnp.zeros((_BT, _D_OUT), jnp.float32)
        )


def kernel(x, lora_a, lora_b, slot_ids):
    slot_ids = slot_ids.astype(jnp.int32)
    a_t = jnp.swapaxes(lora_a.astype(jnp.bfloat16), 1, 2)
    b2d = lora_b.reshape(_NUM_SLOTS * _RANK, _D_OUT)
    grid_spec = pltpu.PrefetchScalarGridSpec(
        num_scalar_prefetch=1,
        grid=(_TOKENS // _BT,),
        in_specs=[
            pl.BlockSpec((_BT, _D_IN), lambda i, s: (i, 0)),
            pl.BlockSpec((_NUM_SLOTS, _RANK, _D_IN), lambda i, s: (0, 0, 0)),
            pl.BlockSpec((_NUM_SLOTS * _RANK, _D_OUT), lambda i, s: (0, 0)),
        ],
        out_specs=pl.BlockSpec((_BT, _D_OUT), lambda i, s: (i, 0)),
        scratch_shapes=[
            pltpu.VMEM((_NUM_SLOTS * _RANK, _D_OUT), jnp.bfloat16),
            pltpu.VMEM((_BT, _W), jnp.bfloat16),
        ],
    )
    return pl.pallas_call(
        _lora_block_kernel,
        grid_spec=grid_spec,
        out_shape=jax.ShapeDtypeStruct((_TOKENS, _D_OUT), jnp.float32),
    )(slot_ids, x, a_t, b2d)
